# Initial kernel scaffold; baseline (speedup 1.0000x reference)
import jax
import jax.numpy as jnp
from jax.experimental import pallas as pl


def kernel(all_timestamps, ts_w, pos_w):
    raise NotImplementedError("write your pallas kernel here")


# speedup vs baseline: 1119.7164x; 1119.7164x over previous
import jax
import jax.numpy as jnp
from jax.experimental import pallas as pl
from jax.experimental.pallas import tpu as pltpu

_B = 1024
_N = 200
_NUM_BUCKETS = 128
_BB = 8


def _body(ts_ref, tsw_ref, posw_ref, out_ref, pos_scratch):
    @pl.when(pl.program_id(0) == 0)
    def _():
        ii = jax.lax.broadcasted_iota(jnp.int32, (_N, _N), 0)
        jj = jax.lax.broadcasted_iota(jnp.int32, (_N, _N), 1)
        v = (_N - 1) + jj - ii
        hi = v >> 7
        lo = v & 127
        acc = jnp.zeros((_N, _N), jnp.float32)
        for k in range(4):
            chunk = jnp.broadcast_to(
                posw_ref[0:1, 128 * k : 128 * (k + 1)], (_N, 128)
            )
            g = jnp.take_along_axis(chunk, lo, axis=1)
            acc = jnp.where(hi == k, g, acc)
        pos_scratch[...] = acc

    ts = ts_ref[...]
    r = jnp.concatenate([ts[:, 1:], ts[:, _N - 1 :]], axis=1)
    diff = r[:, :, None] - ts[:, None, :]
    a = jnp.maximum(jnp.abs(diff).astype(jnp.float32), 1.0)
    y = jnp.log(a) / 0.301
    bucket = y.astype(jnp.int32)
    table = jnp.broadcast_to(tsw_ref[0, :].reshape(1, 1, _NUM_BUCKETS),
                             (_BB, _N, _NUM_BUCKETS))
    ts_bias = jnp.take_along_axis(table, bucket, axis=2)
    out_ref[...] = ts_bias + pos_scratch[...][None, :, :]


def kernel(all_timestamps, ts_w, pos_w):
    tsw = ts_w[: _NUM_BUCKETS].reshape(1, _NUM_BUCKETS)
    posw = jnp.pad(pos_w, (0, 512 - (2 * _N - 1))).reshape(1, 512)
    grid = (_B // _BB,)
    return pl.pallas_call(
        _body,
        grid=grid,
        in_specs=[
            pl.BlockSpec((_BB, _N), lambda i: (i, 0)),
            pl.BlockSpec((1, _NUM_BUCKETS), lambda i: (0, 0)),
            pl.BlockSpec((1, 512), lambda i: (0, 0)),
        ],
        out_specs=pl.BlockSpec((_BB, _N, _N), lambda i: (i, 0, 0)),
        out_shape=jax.ShapeDtypeStruct((_B, _N, _N), jnp.float32),
        scratch_shapes=[pltpu.VMEM((_N, _N), jnp.float32)],
    )(all_timestamps, tsw, posw)


# speedup vs baseline: 1172.0811x; 1.0468x over previous
import jax
import jax.numpy as jnp
from jax.experimental import pallas as pl
from jax.experimental.pallas import tpu as pltpu

_B = 1024
_N = 200
_NUM_BUCKETS = 128
_BB = 64


def _pos_body(posw_ref, out_ref):
    ii = jax.lax.broadcasted_iota(jnp.int32, (_N, _N), 0)
    jj = jax.lax.broadcasted_iota(jnp.int32, (_N, _N), 1)
    v = (_N - 1) + jj - ii
    hi = v >> 7
    lo = v & 127
    acc = jnp.zeros((_N, _N), jnp.float32)
    for k in range(4):
        chunk = jnp.broadcast_to(posw_ref[0:1, 128 * k : 128 * (k + 1)], (_N, 128))
        g = jnp.take_along_axis(chunk, lo, axis=1)
        acc = jnp.where(hi == k, g, acc)
    out_ref[...] = acc


def _main_body(ts_ref, tsw_ref, pos_ref, out_ref):
    ts = ts_ref[...]
    r = jnp.concatenate([ts[:, 1:], ts[:, _N - 1 :]], axis=1)
    diff = (r[:, :, None] - ts[:, None, :]).astype(jnp.float32)
    a = jnp.maximum(jnp.abs(diff), 1.0)
    y = jnp.log(a) / 0.301
    bucket = y.astype(jnp.int32)
    table = jnp.broadcast_to(
        tsw_ref[0:1, :].reshape(1, 1, _NUM_BUCKETS), (_BB, _N, _NUM_BUCKETS)
    )
    ts_bias = jnp.take_along_axis(table, bucket, axis=2)
    out_ref[...] = ts_bias + pos_ref[...][None, :, :]


def kernel(all_timestamps, ts_w, pos_w):
    tsw = jnp.broadcast_to(ts_w[: _NUM_BUCKETS].reshape(1, _NUM_BUCKETS), (8, _NUM_BUCKETS))
    posw = jnp.pad(pos_w, (0, 512 - (2 * _N - 1))).reshape(1, 512)

    pos_mat = pl.pallas_call(
        _pos_body,
        out_shape=jax.ShapeDtypeStruct((_N, _N), jnp.float32),
    )(posw)

    return pl.pallas_call(
        _main_body,
        grid=(_B // _BB,),
        in_specs=[
            pl.BlockSpec((_BB, _N), lambda i: (i, 0)),
            pl.BlockSpec((8, _NUM_BUCKETS), lambda i: (0, 0)),
            pl.BlockSpec((_N, _N), lambda i: (0, 0)),
        ],
        out_specs=pl.BlockSpec((_BB, _N, _N), lambda i: (i, 0, 0)),
        out_shape=jax.ShapeDtypeStruct((_B, _N, _N), jnp.float32),
    )(all_timestamps, tsw, pos_mat)
